# Initial kernel scaffold; baseline (speedup 1.0000x reference)
import jax
import jax.numpy as jnp
from jax.experimental import pallas as pl


def kernel(x, edge_index, W0, b0, W1, b1, W2, b2):
    raise NotImplementedError("write your pallas kernel here")


# speedup vs baseline: 35.8814x; 35.8814x over previous
import functools

import jax
import jax.numpy as jnp
from jax import lax
from jax.experimental import pallas as pl
from jax.experimental.pallas import tpu as pltpu
from jax.experimental.pallas import tpu_sc as plsc

NC = 2
NS = 16
NW = NC * NS
SUB = 128
CHUNK = 2048
F = 8


def _ceil_to(x, m):
    return (x + m - 1) // m * m


@functools.lru_cache(maxsize=None)
def _make_agg_kernel(n_pad, e_pad, gather: bool):
    rows_per_tile = n_pad // NS
    chunks_per_worker = e_pad // (NW * CHUNK)
    mesh = plsc.VectorSubcoreMesh(core_axis_name="c", subcore_axis_name="s")

    def body(src2d, dst2d, y, zeros, ones, out, src_v, dst_v, rows_v, acc, sem):
        c = lax.axis_index("c")
        s = lax.axis_index("s")
        wid = s * NC + c
        t0 = s * rows_per_tile
        pltpu.sync_copy(zeros.at[pl.ds(t0, rows_per_tile)],
                        acc.at[pl.ds(t0, rows_per_tile)])
        if not gather:
            pltpu.sync_copy(ones, rows_v)
        plsc.subcore_barrier()

        base = wid * chunks_per_worker * (CHUNK // SUB)

        def chunk_body(ci, carry):
            r0 = base + ci * (CHUNK // SUB)
            if gather:
                pltpu.sync_copy(src2d.at[pl.ds(r0, CHUNK // SUB)], src_v)
            pltpu.sync_copy(dst2d.at[pl.ds(r0, CHUNK // SUB)], dst_v)
            if gather:
                handles = []
                for j in range(CHUNK // SUB):
                    handles.append(
                        pltpu.async_copy(y.at[src_v.at[j]],
                                         rows_v.at[pl.ds(j * SUB, SUB)], sem))
                for h in handles:
                    h.wait()
            for j in range(CHUNK // SUB):
                pltpu.sync_copy(rows_v.at[pl.ds(j * SUB, SUB)],
                                acc.at[dst_v.at[j]], add=True)
            return carry

        lax.fori_loop(0, chunks_per_worker, chunk_body, 0)
        plsc.subcore_barrier()
        pltpu.sync_copy(acc.at[pl.ds(t0, rows_per_tile)],
                        out.at[c, pl.ds(t0, rows_per_tile)])

    return pl.kernel(
        body,
        out_type=jax.ShapeDtypeStruct((NC, n_pad, F), jnp.float32),
        mesh=mesh,
        scratch_types=[
            pltpu.VMEM((CHUNK // SUB, SUB), jnp.int32),
            pltpu.VMEM((CHUNK // SUB, SUB), jnp.int32),
            pltpu.VMEM((CHUNK, F), jnp.float32),
            pltpu.VMEM_SHARED((n_pad, F), jnp.float32),
            pltpu.SemaphoreType.DMA,
        ],
        compiler_params=pltpu.CompilerParams(use_tc_tiling_on_sc=False),
    )


_RB = 512


def _row_spec():
    return pl.BlockSpec((_RB, F), lambda i: (i, 0))


def _w_spec():
    return pl.BlockSpec((F, F), lambda i: (0, 0))


def _b_spec():
    return pl.BlockSpec((1, F), lambda i: (0, 0))


def _tc0_body(degA, degB, x8, dis_o, q_o, xs_o):
    deg = degA[...] + degB[...] + 1.0
    dis = lax.rsqrt(deg)
    dis_o[...] = dis
    q_o[...] = 1.0 / deg
    xs_o[...] = x8[...] * dis


def _tc1_body(aggA, aggB, x8, dis, q, w0l, w0h, w1ll, w1lh, w1hl, w1hh,
              b0l, b0h, h1l_o, h1h_o, y1l_o, y1h_o):
    ax = dis[...] * (aggA[...] + aggB[...]) + q[...] * x8[...]
    o0l = jnp.maximum(jnp.dot(ax, w0l[...],
                              preferred_element_type=jnp.float32) + b0l[...], 0.0)
    o0h = jnp.maximum(jnp.dot(ax, w0h[...],
                              preferred_element_type=jnp.float32) + b0h[...], 0.0)
    h1l = (jnp.dot(o0l, w1ll[...], preferred_element_type=jnp.float32)
           + jnp.dot(o0h, w1hl[...], preferred_element_type=jnp.float32))
    h1h = (jnp.dot(o0l, w1lh[...], preferred_element_type=jnp.float32)
           + jnp.dot(o0h, w1hh[...], preferred_element_type=jnp.float32))
    h1l_o[...] = h1l
    h1h_o[...] = h1h
    y1l_o[...] = h1l * dis[...]
    y1h_o[...] = h1h * dis[...]


def _tc2_body(aggLA, aggLB, aggHA, aggHB, h1l, h1h, dis, q, w2l, w2h,
              b1l, b1h, h2_o, y2_o):
    o1l = jnp.maximum(dis[...] * (aggLA[...] + aggLB[...])
                      + q[...] * h1l[...] + b1l[...], 0.0)
    o1h = jnp.maximum(dis[...] * (aggHA[...] + aggHB[...])
                      + q[...] * h1h[...] + b1h[...], 0.0)
    h2 = (jnp.dot(o1l, w2l[...], preferred_element_type=jnp.float32)
          + jnp.dot(o1h, w2h[...], preferred_element_type=jnp.float32))
    h2_o[...] = h2
    y2_o[...] = h2 * dis[...]


def _tc3_body(aggA, aggB, h2, dis, q, b2, out_o):
    z = dis[...] * (aggA[...] + aggB[...]) + q[...] * h2[...] + b2[...]
    out_o[...] = jax.nn.sigmoid(z)


@functools.lru_cache(maxsize=None)
def _make_tc_kernels(n_pad):
    rs = jax.ShapeDtypeStruct((n_pad, F), jnp.float32)
    grid = (n_pad // _RB,)
    r, w, b = _row_spec, _w_spec, _b_spec

    tc0 = pl.pallas_call(
        _tc0_body, grid=grid, in_specs=[r(), r(), r()],
        out_specs=[r()] * 3, out_shape=[rs] * 3)
    tc1 = pl.pallas_call(
        _tc1_body, grid=grid,
        in_specs=[r()] * 5 + [w()] * 6 + [b()] * 2,
        out_specs=[r()] * 4, out_shape=[rs] * 4)
    tc2 = pl.pallas_call(
        _tc2_body, grid=grid,
        in_specs=[r()] * 8 + [w()] * 2 + [b()] * 2,
        out_specs=[r()] * 2, out_shape=[rs] * 2)
    tc3 = pl.pallas_call(
        _tc3_body, grid=grid, in_specs=[r()] * 5 + [b()],
        out_specs=r(), out_shape=rs)
    return tc0, tc1, tc2, tc3


def kernel(x, edge_index, W0, b0, W1, b1, W2, b2):
    n = x.shape[0]
    e = edge_index.shape[1]
    n_pad = _ceil_to(n + 1, CHUNK)
    e_pad = _ceil_to(e, NW * CHUNK)

    src = jnp.concatenate(
        [edge_index[0], jnp.full((e_pad - e,), n, jnp.int32)]).reshape(-1, SUB)
    dst = jnp.concatenate(
        [edge_index[1], jnp.full((e_pad - e,), n, jnp.int32)]).reshape(-1, SUB)
    x8 = jnp.pad(x, ((0, n_pad - n), (0, F - x.shape[1])))
    w0p = jnp.pad(W0, ((0, F - W0.shape[0]), (0, 0)))
    w0l, w0h = w0p[:, :F], w0p[:, F:]
    w1ll, w1lh = W1[:F, :F], W1[:F, F:]
    w1hl, w1hh = W1[F:, :F], W1[F:, F:]
    w2p = jnp.pad(W2, ((0, 0), (0, F - W2.shape[1])))
    w2l, w2h = w2p[:F], w2p[F:]
    b0l, b0h = b0[:F].reshape(1, F), b0[F:].reshape(1, F)
    b1l, b1h = b1[:F].reshape(1, F), b1[F:].reshape(1, F)
    b2p = jnp.pad(b2, (0, F - b2.shape[0])).reshape(1, F)
    zeros = jnp.zeros((n_pad, F), jnp.float32)
    ones = jnp.ones((CHUNK, F), jnp.float32)

    deg_pass = _make_agg_kernel(n_pad, e_pad, gather=False)
    agg_pass = _make_agg_kernel(n_pad, e_pad, gather=True)
    tc0, tc1, tc2, tc3 = _make_tc_kernels(n_pad)

    deg_parts = deg_pass(src, dst, zeros, zeros, ones)
    dis, q, xs = tc0(deg_parts[0], deg_parts[1], x8)

    agg0 = agg_pass(src, dst, xs, zeros, ones)
    h1l, h1h, y1l, y1h = tc1(agg0[0], agg0[1], x8, dis, q,
                             w0l, w0h, w1ll, w1lh, w1hl, w1hh, b0l, b0h)

    agg1l = agg_pass(src, dst, y1l, zeros, ones)
    agg1h = agg_pass(src, dst, y1h, zeros, ones)
    h2, y2 = tc2(agg1l[0], agg1l[1], agg1h[0], agg1h[1], h1l, h1h, dis, q,
                 w2l, w2h, b1l, b1h)

    agg2 = agg_pass(src, dst, y2, zeros, ones)
    out = tc3(agg2[0], agg2[1], h2, dis, q, b2p)

    return out[:n, 0:1]


# speedup vs baseline: 63.9544x; 1.7824x over previous
import functools

import jax
import jax.numpy as jnp
from jax import lax
from jax.experimental import pallas as pl
from jax.experimental.pallas import tpu as pltpu
from jax.experimental.pallas import tpu_sc as plsc

NC = 2
NS = 16
NW = NC * NS
SUB = 128
CHUNK = 2048
F = 8
LANES = 128
NPG = LANES // F


def _ceil_to(x, m):
    return (x + m - 1) // m * m


@functools.lru_cache(maxsize=None)
def _make_agg_kernel(n_pad, e_pad, gather: bool):
    rows_per_tile = n_pad // NS
    chunks_per_worker = e_pad // (NW * CHUNK)
    mesh = plsc.VectorSubcoreMesh(core_axis_name="c", subcore_axis_name="s")

    def body(src2d, dst2d, y, zeros, ones, out, src_v, dst_v, rows_v, acc, sem):
        c = lax.axis_index("c")
        s = lax.axis_index("s")
        wid = s * NC + c
        t0 = s * rows_per_tile
        pltpu.sync_copy(zeros.at[pl.ds(t0, rows_per_tile)],
                        acc.at[pl.ds(t0, rows_per_tile)])
        if not gather:
            pltpu.sync_copy(ones, rows_v)
        plsc.subcore_barrier()

        base = wid * chunks_per_worker * (CHUNK // SUB)

        def chunk_body(ci, carry):
            r0 = base + ci * (CHUNK // SUB)
            if gather:
                pltpu.sync_copy(src2d.at[pl.ds(r0, CHUNK // SUB)], src_v)
            pltpu.sync_copy(dst2d.at[pl.ds(r0, CHUNK // SUB)], dst_v)
            if gather:
                handles = []
                for j in range(CHUNK // SUB):
                    handles.append(
                        pltpu.async_copy(y.at[src_v.at[j]],
                                         rows_v.at[pl.ds(j * SUB, SUB)], sem))
                for h in handles:
                    h.wait()
            for j in range(CHUNK // SUB):
                pltpu.sync_copy(rows_v.at[pl.ds(j * SUB, SUB)],
                                acc.at[dst_v.at[j]], add=True)
            return carry

        lax.fori_loop(0, chunks_per_worker, chunk_body, 0)
        plsc.subcore_barrier()
        pltpu.sync_copy(acc.at[pl.ds(t0, rows_per_tile)],
                        out.at[c, pl.ds(t0, rows_per_tile)])

    return pl.kernel(
        body,
        out_type=jax.ShapeDtypeStruct((NC, n_pad, F), jnp.float32),
        mesh=mesh,
        scratch_types=[
            pltpu.VMEM((CHUNK // SUB, SUB), jnp.int32),
            pltpu.VMEM((CHUNK // SUB, SUB), jnp.int32),
            pltpu.VMEM((CHUNK, F), jnp.float32),
            pltpu.VMEM_SHARED((n_pad, F), jnp.float32),
            pltpu.SemaphoreType.DMA,
        ],
        compiler_params=pltpu.CompilerParams(use_tc_tiling_on_sc=False),
    )


_GB = 448


def _r():
    return pl.BlockSpec((_GB, LANES), lambda i: (i, 0))


def _p():
    return pl.BlockSpec((NC, _GB, LANES), lambda i: (0, i, 0))


def _w():
    return pl.BlockSpec((LANES, LANES), lambda i: (0, 0))


def _b():
    return pl.BlockSpec((1, LANES), lambda i: (0, 0))


def _mm(a, w):
    return jnp.dot(a, w[...], preferred_element_type=jnp.float32)


def _tc0_body(degp, x8, dis_o, q_o, xs_o):
    deg = degp[0] + degp[1] + 1.0
    dis = lax.rsqrt(deg)
    dis_o[...] = dis
    q_o[...] = 1.0 / deg
    xs_o[...] = x8[...] * dis


def _tc1_body(aggp, x8, dis, q, w0l, w0h, w1ll, w1lh, w1hl, w1hh,
              b0l, b0h, h1l_o, h1h_o, y1l_o, y1h_o):
    ax = dis[...] * (aggp[0] + aggp[1]) + q[...] * x8[...]
    o0l = jnp.maximum(_mm(ax, w0l) + b0l[...], 0.0)
    o0h = jnp.maximum(_mm(ax, w0h) + b0h[...], 0.0)
    h1l = _mm(o0l, w1ll) + _mm(o0h, w1hl)
    h1h = _mm(o0l, w1lh) + _mm(o0h, w1hh)
    h1l_o[...] = h1l
    h1h_o[...] = h1h
    y1l_o[...] = h1l * dis[...]
    y1h_o[...] = h1h * dis[...]


def _tc2_body(aggpl, aggph, h1l, h1h, dis, q, w2l, w2h, b1l, b1h, h2_o, y2_o):
    o1l = jnp.maximum(dis[...] * (aggpl[0] + aggpl[1])
                      + q[...] * h1l[...] + b1l[...], 0.0)
    o1h = jnp.maximum(dis[...] * (aggph[0] + aggph[1])
                      + q[...] * h1h[...] + b1h[...], 0.0)
    h2 = _mm(o1l, w2l) + _mm(o1h, w2h)
    h2_o[...] = h2
    y2_o[...] = h2 * dis[...]


def _tc3_body(aggp, h2, dis, q, b2, out_o):
    z = dis[...] * (aggp[0] + aggp[1]) + q[...] * h2[...] + b2[...]
    out_o[...] = jax.nn.sigmoid(z)


@functools.lru_cache(maxsize=None)
def _make_tc_kernels(g):
    rs = jax.ShapeDtypeStruct((g, LANES), jnp.float32)
    grid = (g // _GB,)
    r, p, w, b = _r, _p, _w, _b

    tc0 = pl.pallas_call(
        _tc0_body, grid=grid, in_specs=[p(), r()],
        out_specs=[r()] * 3, out_shape=[rs] * 3)
    tc1 = pl.pallas_call(
        _tc1_body, grid=grid,
        in_specs=[p()] + [r()] * 3 + [w()] * 6 + [b()] * 2,
        out_specs=[r()] * 4, out_shape=[rs] * 4)
    tc2 = pl.pallas_call(
        _tc2_body, grid=grid,
        in_specs=[p(), p()] + [r()] * 4 + [w()] * 2 + [b()] * 2,
        out_specs=[r()] * 2, out_shape=[rs] * 2)
    tc3 = pl.pallas_call(
        _tc3_body, grid=grid, in_specs=[p()] + [r()] * 3 + [b()],
        out_specs=r(), out_shape=rs)
    return tc0, tc1, tc2, tc3


def kernel(x, edge_index, W0, b0, W1, b1, W2, b2):
    n = x.shape[0]
    e = edge_index.shape[1]
    n_pad = _ceil_to(n + 1, CHUNK)
    g = n_pad * F // LANES

    e_pad = _ceil_to(e, NW * CHUNK)

    src = jnp.concatenate(
        [edge_index[0], jnp.full((e_pad - e,), n, jnp.int32)]).reshape(-1, SUB)
    dst = jnp.concatenate(
        [edge_index[1], jnp.full((e_pad - e,), n, jnp.int32)]).reshape(-1, SUB)
    x8 = jnp.pad(x, ((0, n_pad - n), (0, F - x.shape[1]))).reshape(g, LANES)

    eye = jnp.eye(NPG, dtype=jnp.float32)

    def bd(w8):
        return jnp.kron(eye, w8)

    w0p = jnp.pad(W0, ((0, F - W0.shape[0]), (0, 0)))
    w0l, w0h = bd(w0p[:, :F]), bd(w0p[:, F:])
    w1ll, w1lh = bd(W1[:F, :F]), bd(W1[:F, F:])
    w1hl, w1hh = bd(W1[F:, :F]), bd(W1[F:, F:])
    w2p = jnp.pad(W2, ((0, 0), (0, F - W2.shape[1])))
    w2l, w2h = bd(w2p[:F]), bd(w2p[F:])
    b0l, b0h = jnp.tile(b0[:F], NPG).reshape(1, LANES), \
        jnp.tile(b0[F:], NPG).reshape(1, LANES)
    b1l, b1h = jnp.tile(b1[:F], NPG).reshape(1, LANES), \
        jnp.tile(b1[F:], NPG).reshape(1, LANES)
    b2p = jnp.tile(jnp.pad(b2, (0, F - b2.shape[0])), NPG).reshape(1, LANES)
    zeros = jnp.zeros((n_pad, F), jnp.float32)
    ones = jnp.ones((CHUNK, F), jnp.float32)

    deg_pass = _make_agg_kernel(n_pad, e_pad, gather=False)
    agg_pass = _make_agg_kernel(n_pad, e_pad, gather=True)
    tc0, tc1, tc2, tc3 = _make_tc_kernels(g)

    def flat(a):
        return a.reshape(NC, g, LANES)

    def tab(a):
        return a.reshape(n_pad, F)

    deg_parts = deg_pass(src, dst, zeros, zeros, ones)
    dis, q, xs = tc0(flat(deg_parts), x8)

    agg0 = agg_pass(src, dst, tab(xs), zeros, ones)
    h1l, h1h, y1l, y1h = tc1(flat(agg0), x8, dis, q,
                             w0l, w0h, w1ll, w1lh, w1hl, w1hh, b0l, b0h)

    agg1l = agg_pass(src, dst, tab(y1l), zeros, ones)
    agg1h = agg_pass(src, dst, tab(y1h), zeros, ones)
    h2, y2 = tc2(flat(agg1l), flat(agg1h), h1l, h1h, dis, q,
                 w2l, w2h, b1l, b1h)

    agg2 = agg_pass(src, dst, tab(y2), zeros, ones)
    out = tc3(flat(agg2), h2, dis, q, b2p)

    return out.reshape(n_pad, F)[:n, 0:1]


# speedup vs baseline: 86.2045x; 1.3479x over previous
import functools

import jax
import jax.numpy as jnp
from jax import lax
from jax.experimental import pallas as pl
from jax.experimental.pallas import tpu as pltpu
from jax.experimental.pallas import tpu_sc as plsc

NC = 2
NS = 16
NW = NC * NS
SUB = 128
CHUNK = 1024
F = 8
LANES = 128
NPG = LANES // F


def _ceil_to(x, m):
    return (x + m - 1) // m * m


@functools.lru_cache(maxsize=None)
def _make_agg_kernel(n_pad, e_pad, gather: bool):
    rows_per_tile = n_pad // NS
    chunks_per_worker = e_pad // (NW * CHUNK)
    mesh = plsc.VectorSubcoreMesh(core_axis_name="c", subcore_axis_name="s")

    ncw = chunks_per_worker
    nsb = CHUNK // SUB

    def body(src2d, dst2d, y, zeros, ones, out,
             src_v, dst_v, rows_v, acc, gsem, isem):
        c = lax.axis_index("c")
        s = lax.axis_index("s")
        wid = s * NC + c
        t0 = s * rows_per_tile
        pltpu.sync_copy(zeros.at[pl.ds(t0, rows_per_tile)],
                        acc.at[pl.ds(t0, rows_per_tile)])
        if not gather:
            pltpu.sync_copy(ones, rows_v.at[0])
            pltpu.sync_copy(ones, rows_v.at[1])
        plsc.subcore_barrier()

        base = wid * ncw * nsb

        def load_idx(ck, p, sync):
            r0 = base + ck * nsb
            if sync:
                if gather:
                    pltpu.sync_copy(src2d.at[pl.ds(r0, nsb)], src_v.at[p])
                pltpu.sync_copy(dst2d.at[pl.ds(r0, nsb)], dst_v.at[p])
            else:
                if gather:
                    pltpu.async_copy(src2d.at[pl.ds(r0, nsb)], src_v.at[p],
                                     isem.at[p])
                pltpu.async_copy(dst2d.at[pl.ds(r0, nsb)], dst_v.at[p],
                                 isem.at[p])

        def wait_idx(p):
            if gather:
                pltpu.make_async_copy(src2d.at[pl.ds(0, nsb)], src_v.at[p],
                                      isem.at[p]).wait()
            pltpu.make_async_copy(dst2d.at[pl.ds(0, nsb)], dst_v.at[p],
                                  isem.at[p]).wait()

        def issue_gathers(p):
            for j in range(nsb):
                pltpu.async_copy(y.at[src_v.at[p, j]],
                                 rows_v.at[p, pl.ds(j * SUB, SUB)],
                                 gsem.at[p])

        def wait_gathers(p):
            for j in range(nsb):
                pltpu.make_async_copy(y.at[src_v.at[p, j]],
                                      rows_v.at[p, pl.ds(j * SUB, SUB)],
                                      gsem.at[p]).wait()

        def scatters(p):
            for j in range(nsb):
                pltpu.sync_copy(rows_v.at[p, pl.ds(j * SUB, SUB)],
                                acc.at[dst_v.at[p, j]], add=True)

        load_idx(0, 0, sync=True)
        if gather:
            issue_gathers(0)
        load_idx(1, 1, sync=False)

        def step(i, carry):
            for ph in range(2):
                ck = i * 2 + ph
                p = ph
                q = 1 - ph

                @pl.when(ck + 1 < ncw)
                def _():
                    wait_idx(q)
                    if gather:
                        issue_gathers(q)

                if gather:
                    wait_gathers(p)
                scatters(p)

                @pl.when(ck + 2 < ncw)
                def _():
                    load_idx(ck + 2, p, sync=False)
            return carry

        lax.fori_loop(0, ncw // 2, step, 0)
        plsc.subcore_barrier()
        pltpu.sync_copy(acc.at[pl.ds(t0, rows_per_tile)],
                        out.at[c, pl.ds(t0, rows_per_tile)])

    return pl.kernel(
        body,
        out_type=jax.ShapeDtypeStruct((NC, n_pad, F), jnp.float32),
        mesh=mesh,
        scratch_types=[
            pltpu.VMEM((2, CHUNK // SUB, SUB), jnp.int32),
            pltpu.VMEM((2, CHUNK // SUB, SUB), jnp.int32),
            pltpu.VMEM((2, CHUNK, F), jnp.float32),
            pltpu.VMEM_SHARED((n_pad, F), jnp.float32),
            pltpu.SemaphoreType.DMA((2,)),
            pltpu.SemaphoreType.DMA((2,)),
        ],
        compiler_params=pltpu.CompilerParams(use_tc_tiling_on_sc=False),
    )


_GB = 448


def _r():
    return pl.BlockSpec((_GB, LANES), lambda i: (i, 0))


def _p():
    return pl.BlockSpec((NC, _GB, LANES), lambda i: (0, i, 0))


def _w():
    return pl.BlockSpec((LANES, LANES), lambda i: (0, 0))


def _b():
    return pl.BlockSpec((1, LANES), lambda i: (0, 0))


def _mm(a, w):
    return jnp.dot(a, w[...], preferred_element_type=jnp.float32)


def _tc0_body(degp, x8, dis_o, q_o, xs_o):
    deg = degp[0] + degp[1] + 1.0
    dis = lax.rsqrt(deg)
    dis_o[...] = dis
    q_o[...] = 1.0 / deg
    xs_o[...] = x8[...] * dis


def _tc1_body(aggp, x8, dis, q, w0l, w0h, w1ll, w1lh, w1hl, w1hh,
              b0l, b0h, h1l_o, h1h_o, y1l_o, y1h_o):
    ax = dis[...] * (aggp[0] + aggp[1]) + q[...] * x8[...]
    o0l = jnp.maximum(_mm(ax, w0l) + b0l[...], 0.0)
    o0h = jnp.maximum(_mm(ax, w0h) + b0h[...], 0.0)
    h1l = _mm(o0l, w1ll) + _mm(o0h, w1hl)
    h1h = _mm(o0l, w1lh) + _mm(o0h, w1hh)
    h1l_o[...] = h1l
    h1h_o[...] = h1h
    y1l_o[...] = h1l * dis[...]
    y1h_o[...] = h1h * dis[...]


def _tc2_body(aggpl, aggph, h1l, h1h, dis, q, w2l, w2h, b1l, b1h, h2_o, y2_o):
    o1l = jnp.maximum(dis[...] * (aggpl[0] + aggpl[1])
                      + q[...] * h1l[...] + b1l[...], 0.0)
    o1h = jnp.maximum(dis[...] * (aggph[0] + aggph[1])
                      + q[...] * h1h[...] + b1h[...], 0.0)
    h2 = _mm(o1l, w2l) + _mm(o1h, w2h)
    h2_o[...] = h2
    y2_o[...] = h2 * dis[...]


def _tc3_body(aggp, h2, dis, q, b2, out_o):
    z = dis[...] * (aggp[0] + aggp[1]) + q[...] * h2[...] + b2[...]
    out_o[...] = jax.nn.sigmoid(z)


@functools.lru_cache(maxsize=None)
def _make_tc_kernels(g):
    rs = jax.ShapeDtypeStruct((g, LANES), jnp.float32)
    grid = (g // _GB,)
    r, p, w, b = _r, _p, _w, _b

    tc0 = pl.pallas_call(
        _tc0_body, grid=grid, in_specs=[p(), r()],
        out_specs=[r()] * 3, out_shape=[rs] * 3)
    tc1 = pl.pallas_call(
        _tc1_body, grid=grid,
        in_specs=[p()] + [r()] * 3 + [w()] * 6 + [b()] * 2,
        out_specs=[r()] * 4, out_shape=[rs] * 4)
    tc2 = pl.pallas_call(
        _tc2_body, grid=grid,
        in_specs=[p(), p()] + [r()] * 4 + [w()] * 2 + [b()] * 2,
        out_specs=[r()] * 2, out_shape=[rs] * 2)
    tc3 = pl.pallas_call(
        _tc3_body, grid=grid, in_specs=[p()] + [r()] * 3 + [b()],
        out_specs=r(), out_shape=rs)
    return tc0, tc1, tc2, tc3


def kernel(x, edge_index, W0, b0, W1, b1, W2, b2):
    n = x.shape[0]
    e = edge_index.shape[1]
    n_pad = _ceil_to(n + 1, CHUNK)
    g = n_pad * F // LANES

    e_pad = _ceil_to(e, 2 * NW * CHUNK)

    src = jnp.concatenate(
        [edge_index[0], jnp.full((e_pad - e,), n, jnp.int32)]).reshape(-1, SUB)
    dst = jnp.concatenate(
        [edge_index[1], jnp.full((e_pad - e,), n, jnp.int32)]).reshape(-1, SUB)
    x8 = jnp.pad(x, ((0, n_pad - n), (0, F - x.shape[1]))).reshape(g, LANES)

    eye = jnp.eye(NPG, dtype=jnp.float32)

    def bd(w8):
        return jnp.kron(eye, w8)

    w0p = jnp.pad(W0, ((0, F - W0.shape[0]), (0, 0)))
    w0l, w0h = bd(w0p[:, :F]), bd(w0p[:, F:])
    w1ll, w1lh = bd(W1[:F, :F]), bd(W1[:F, F:])
    w1hl, w1hh = bd(W1[F:, :F]), bd(W1[F:, F:])
    w2p = jnp.pad(W2, ((0, 0), (0, F - W2.shape[1])))
    w2l, w2h = bd(w2p[:F]), bd(w2p[F:])
    b0l, b0h = jnp.tile(b0[:F], NPG).reshape(1, LANES), \
        jnp.tile(b0[F:], NPG).reshape(1, LANES)
    b1l, b1h = jnp.tile(b1[:F], NPG).reshape(1, LANES), \
        jnp.tile(b1[F:], NPG).reshape(1, LANES)
    b2p = jnp.tile(jnp.pad(b2, (0, F - b2.shape[0])), NPG).reshape(1, LANES)
    zeros = jnp.zeros((n_pad, F), jnp.float32)
    ones = jnp.ones((CHUNK, F), jnp.float32)

    deg_pass = _make_agg_kernel(n_pad, e_pad, gather=False)
    agg_pass = _make_agg_kernel(n_pad, e_pad, gather=True)
    tc0, tc1, tc2, tc3 = _make_tc_kernels(g)

    def flat(a):
        return a.reshape(NC, g, LANES)

    def tab(a):
        return a.reshape(n_pad, F)

    deg_parts = deg_pass(src, dst, zeros, zeros, ones)
    dis, q, xs = tc0(flat(deg_parts), x8)

    agg0 = agg_pass(src, dst, tab(xs), zeros, ones)
    h1l, h1h, y1l, y1h = tc1(flat(agg0), x8, dis, q,
                             w0l, w0h, w1ll, w1lh, w1hl, w1hh, b0l, b0h)

    agg1l = agg_pass(src, dst, tab(y1l), zeros, ones)
    agg1h = agg_pass(src, dst, tab(y1h), zeros, ones)
    h2, y2 = tc2(flat(agg1l), flat(agg1h), h1l, h1h, dis, q,
                 w2l, w2h, b1l, b1h)

    agg2 = agg_pass(src, dst, tab(y2), zeros, ones)
    out = tc3(flat(agg2), h2, dis, q, b2p)

    return out.reshape(n_pad, F)[:n, 0:1]


# speedup vs baseline: 102.9541x; 1.1943x over previous
import functools

import jax
import jax.numpy as jnp
from jax import lax
from jax.experimental import pallas as pl
from jax.experimental.pallas import tpu as pltpu
from jax.experimental.pallas import tpu_sc as plsc

NC = 2
NS = 16
NW = NC * NS
SUB = 128
CHUNK = 1024
NSB = CHUNK // SUB
F = 8
LANES = 128
NPG = LANES // F


def _ceil_to(x, m):
    return (x + m - 1) // m * m


@functools.lru_cache(maxsize=None)
def _make_agg_kernel(n_pad, e_pad, gather: bool):
    rows_per_tile = n_pad // NS
    ncw = e_pad // (NW * CHUNK)
    mesh = plsc.VectorSubcoreMesh(core_axis_name="c", subcore_axis_name="s")

    def body(ei3, y, zeros, ones, out,
             src_v, dst_v, rows_v, acc, gsem, isem, ssem):
        c = lax.axis_index("c")
        s = lax.axis_index("s")
        wid = s * NC + c
        t0 = s * rows_per_tile
        pltpu.sync_copy(zeros.at[pl.ds(t0, rows_per_tile)],
                        acc.at[pl.ds(t0, rows_per_tile)])
        if not gather:
            pltpu.sync_copy(ones, rows_v.at[0])
            pltpu.sync_copy(ones, rows_v.at[1])
        plsc.subcore_barrier()

        base = wid * ncw * NSB

        def load_idx(ck, p, sync):
            r0 = base + ck * NSB
            if sync:
                if gather:
                    pltpu.sync_copy(ei3.at[0, pl.ds(r0, NSB)], src_v.at[p])
                pltpu.sync_copy(ei3.at[1, pl.ds(r0, NSB)], dst_v.at[p])
            else:
                if gather:
                    pltpu.async_copy(ei3.at[0, pl.ds(r0, NSB)], src_v.at[p],
                                     isem.at[p])
                pltpu.async_copy(ei3.at[1, pl.ds(r0, NSB)], dst_v.at[p],
                                 isem.at[p])

        def wait_idx(p):
            if gather:
                pltpu.make_async_copy(ei3.at[0, pl.ds(0, NSB)], src_v.at[p],
                                      isem.at[p]).wait()
            pltpu.make_async_copy(ei3.at[1, pl.ds(0, NSB)], dst_v.at[p],
                                  isem.at[p]).wait()

        def issue_gathers(p):
            for j in range(NSB):
                pltpu.async_copy(y.at[src_v.at[p, j]],
                                 rows_v.at[p, pl.ds(j * SUB, SUB)],
                                 gsem.at[p])

        def wait_gathers(p):
            for j in range(NSB):
                pltpu.make_async_copy(y.at[src_v.at[p, j]],
                                      rows_v.at[p, pl.ds(j * SUB, SUB)],
                                      gsem.at[p]).wait()

        def issue_scatters(p):
            for j in range(NSB):
                pltpu.async_copy(rows_v.at[p, pl.ds(j * SUB, SUB)],
                                 acc.at[dst_v.at[p, j]], ssem.at[p], add=True)

        def wait_scatters(p):
            for j in range(NSB):
                pltpu.make_async_copy(rows_v.at[p, pl.ds(j * SUB, SUB)],
                                      acc.at[dst_v.at[p, j]],
                                      ssem.at[p]).wait()

        load_idx(0, 0, sync=True)
        if gather:
            issue_gathers(0)
        load_idx(1, 1, sync=False)

        def step(i, carry):
            for ph in range(2):
                ck = i * 2 + ph
                p = ph
                q = 1 - ph

                @pl.when(ck + 1 < ncw)
                def _():
                    wait_idx(q)
                    if gather:
                        issue_gathers(q)

                if gather:
                    wait_gathers(p)
                issue_scatters(p)
                wait_scatters(p)

                @pl.when(ck + 2 < ncw)
                def _():
                    load_idx(ck + 2, p, sync=False)
            return carry

        lax.fori_loop(0, ncw // 2, step, 0)
        plsc.subcore_barrier()
        pltpu.sync_copy(acc.at[pl.ds(t0, rows_per_tile)],
                        out.at[c, pl.ds(t0, rows_per_tile)])

    return pl.kernel(
        body,
        out_type=jax.ShapeDtypeStruct((NC, n_pad, F), jnp.float32),
        mesh=mesh,
        scratch_types=[
            pltpu.VMEM((2, NSB, SUB), jnp.int32),
            pltpu.VMEM((2, NSB, SUB), jnp.int32),
            pltpu.VMEM((2, CHUNK, F), jnp.float32),
            pltpu.VMEM_SHARED((n_pad, F), jnp.float32),
            pltpu.SemaphoreType.DMA((2,)),
            pltpu.SemaphoreType.DMA((2,)),
            pltpu.SemaphoreType.DMA((2,)),
        ],
        compiler_params=pltpu.CompilerParams(use_tc_tiling_on_sc=False),
    )


_GB = 448


def _r():
    return pl.BlockSpec((_GB, LANES), lambda i: (i, 0))


def _p():
    return pl.BlockSpec((NC, _GB, LANES), lambda i: (0, i, 0))


def _w():
    return pl.BlockSpec((LANES, LANES), lambda i: (0, 0))


def _b():
    return pl.BlockSpec((1, LANES), lambda i: (0, 0))


def _mm(a, w):
    return jnp.dot(a, w[...], preferred_element_type=jnp.float32)


def _tc0_body(degp, x8, dis_o, q_o, xs_o):
    deg = degp[0] + degp[1] + 1.0
    dis = lax.rsqrt(deg)
    dis_o[...] = dis
    q_o[...] = 1.0 / deg
    xs_o[...] = x8[...] * dis


def _tc1_body(aggp, x8, dis, q, w0l, w0h, w1ll, w1lh, w1hl, w1hh,
              b0l, b0h, h1l_o, h1h_o, y1l_o, y1h_o):
    ax = dis[...] * (aggp[0] + aggp[1]) + q[...] * x8[...]
    o0l = jnp.maximum(_mm(ax, w0l) + b0l[...], 0.0)
    o0h = jnp.maximum(_mm(ax, w0h) + b0h[...], 0.0)
    h1l = _mm(o0l, w1ll) + _mm(o0h, w1hl)
    h1h = _mm(o0l, w1lh) + _mm(o0h, w1hh)
    h1l_o[...] = h1l
    h1h_o[...] = h1h
    y1l_o[...] = h1l * dis[...]
    y1h_o[...] = h1h * dis[...]


def _tc2_body(aggpl, aggph, h1l, h1h, dis, q, w2l, w2h, b1l, b1h, h2_o, y2_o):
    o1l = jnp.maximum(dis[...] * (aggpl[0] + aggpl[1])
                      + q[...] * h1l[...] + b1l[...], 0.0)
    o1h = jnp.maximum(dis[...] * (aggph[0] + aggph[1])
                      + q[...] * h1h[...] + b1h[...], 0.0)
    h2 = _mm(o1l, w2l) + _mm(o1h, w2h)
    h2_o[...] = h2
    y2_o[...] = h2 * dis[...]


def _tc3_body(aggp, h2, dis, q, b2, sel, out_o):
    z = dis[...] * (aggp[0] + aggp[1]) + q[...] * h2[...] + b2[...]
    out_o[...] = jnp.dot(jax.nn.sigmoid(z), sel[...],
                         preferred_element_type=jnp.float32,
                         precision=jax.lax.Precision.HIGHEST)


@functools.lru_cache(maxsize=None)
def _make_tc_kernels(g):
    rs = jax.ShapeDtypeStruct((g, LANES), jnp.float32)
    grid = (g // _GB,)
    r, p, w, b = _r, _p, _w, _b

    tc0 = pl.pallas_call(
        _tc0_body, grid=grid, in_specs=[p(), r()],
        out_specs=[r()] * 3, out_shape=[rs] * 3)
    tc1 = pl.pallas_call(
        _tc1_body, grid=grid,
        in_specs=[p()] + [r()] * 3 + [w()] * 6 + [b()] * 2,
        out_specs=[r()] * 4, out_shape=[rs] * 4)
    tc2 = pl.pallas_call(
        _tc2_body, grid=grid,
        in_specs=[p(), p()] + [r()] * 4 + [w()] * 2 + [b()] * 2,
        out_specs=[r()] * 2, out_shape=[rs] * 2)
    sel_spec = pl.BlockSpec((LANES, NPG), lambda i: (0, 0))
    tc3 = pl.pallas_call(
        _tc3_body, grid=grid, in_specs=[p()] + [r()] * 3 + [b(), sel_spec],
        out_specs=pl.BlockSpec((_GB, NPG), lambda i: (i, 0)),
        out_shape=jax.ShapeDtypeStruct((g, NPG), jnp.float32))
    return tc0, tc1, tc2, tc3


def kernel(x, edge_index, W0, b0, W1, b1, W2, b2):
    n = x.shape[0]
    e = edge_index.shape[1]
    n_pad = _ceil_to(n + 1, NS * _GB * LANES // F // NS)
    g = n_pad * F // LANES

    e_pad = _ceil_to(e, 2 * NW * CHUNK)

    if e_pad != e:
        pad = jnp.full((2, e_pad - e), n, jnp.int32)
        ei = jnp.concatenate([edge_index, pad], axis=1)
    else:
        ei = edge_index
    ei3 = ei.reshape(2, e_pad // SUB, SUB)
    x8 = jnp.pad(x, ((0, n_pad - n), (0, F - x.shape[1]))).reshape(g, LANES)

    eye = jnp.eye(NPG, dtype=jnp.float32)

    def bd(w8):
        return jnp.kron(eye, w8)

    w0p = jnp.pad(W0, ((0, F - W0.shape[0]), (0, 0)))
    w0l, w0h = bd(w0p[:, :F]), bd(w0p[:, F:])
    w1ll, w1lh = bd(W1[:F, :F]), bd(W1[:F, F:])
    w1hl, w1hh = bd(W1[F:, :F]), bd(W1[F:, F:])
    w2p = jnp.pad(W2, ((0, 0), (0, F - W2.shape[1])))
    w2l, w2h = bd(w2p[:F]), bd(w2p[F:])
    b0l, b0h = jnp.tile(b0[:F], NPG).reshape(1, LANES), \
        jnp.tile(b0[F:], NPG).reshape(1, LANES)
    b1l, b1h = jnp.tile(b1[:F], NPG).reshape(1, LANES), \
        jnp.tile(b1[F:], NPG).reshape(1, LANES)
    b2p = jnp.tile(jnp.pad(b2, (0, F - b2.shape[0])), NPG).reshape(1, LANES)
    sel = (jnp.arange(LANES)[:, None] == F * jnp.arange(NPG)[None, :]
           ).astype(jnp.float32)
    zeros = jnp.zeros((n_pad, F), jnp.float32)
    ones = jnp.ones((CHUNK, F), jnp.float32)

    deg_pass = _make_agg_kernel(n_pad, e_pad, gather=False)
    agg_pass = _make_agg_kernel(n_pad, e_pad, gather=True)
    tc0, tc1, tc2, tc3 = _make_tc_kernels(g)

    def flat(a):
        return a.reshape(NC, g, LANES)

    def tab(a):
        return a.reshape(n_pad, F)

    deg_parts = deg_pass(ei3, zeros, zeros, ones)
    dis, q, xs = tc0(flat(deg_parts), x8)

    agg0 = agg_pass(ei3, tab(xs), zeros, ones)
    h1l, h1h, y1l, y1h = tc1(flat(agg0), x8, dis, q,
                             w0l, w0h, w1ll, w1lh, w1hl, w1hh, b0l, b0h)

    agg1l = agg_pass(ei3, tab(y1l), zeros, ones)
    agg1h = agg_pass(ei3, tab(y1h), zeros, ones)
    h2, y2 = tc2(flat(agg1l), flat(agg1h), h1l, h1h, dis, q,
                 w2l, w2h, b1l, b1h)

    agg2 = agg_pass(ei3, tab(y2), zeros, ones)
    out = tc3(flat(agg2), h2, dis, q, b2p, sel)

    return out.reshape(n_pad)[:n].reshape(n, 1)
